# Initial kernel scaffold; baseline (speedup 1.0000x reference)
import jax
import jax.numpy as jnp
from jax.experimental import pallas as pl


def kernel(x, As, ycl, enc0_Wg, enc0_bg, enc0_Wc, enc0_bc, enc1_Wg, enc1_bg, enc1_Wc, enc1_bc, dec0_Wg, dec0_bg, dec0_Wc, dec0_bc, dec1_Wg, dec1_bg, dec1_Wc, dec1_bc, Wo, bo):
    raise NotImplementedError("write your pallas kernel here")


# speedup vs baseline: 316.8357x; 316.8357x over previous
import jax
import jax.numpy as jnp
from jax.experimental import pallas as pl

_B = 8
_P = 12
_Q = 12
_N = 512
_E = 8192
_H = 64
_FP = 128
_NMAT = 5


def _adj_kernel(as_ref, a_ref):
    src = as_ref[0, 0, :]
    dst = as_ref[0, 1, :]
    rows = jax.lax.broadcasted_iota(jnp.int32, (_N, _E), 0)
    cols = jax.lax.broadcasted_iota(jnp.int32, (_E, _N), 1)
    dst_oh = (rows == dst[None, :]).astype(jnp.bfloat16)
    src_oh = (cols == src[:, None]).astype(jnp.bfloat16)
    counts = jax.lax.dot_general(
        dst_oh, src_oh, (((1,), (0,)), ((), ())),
        preferred_element_type=jnp.float32)
    deg = jnp.sum(counts, axis=1, keepdims=True)
    a_ref[0] = counts / jnp.maximum(deg, 1.0)


def _build_adj(As):
    return pl.pallas_call(
        _adj_kernel,
        grid=(2,),
        in_specs=[pl.BlockSpec((1, 2, _E), lambda a: (a, 0, 0))],
        out_specs=pl.BlockSpec((1, _N, _N), lambda a: (a, 0, 0)),
        out_shape=jax.ShapeDtypeStruct((2, _N, _N), jnp.float32),
    )(As)


def _prep_w(W, wx):
    F = wx + _H
    O = W.shape[1]
    W5 = W.reshape(_NMAT, F, O)
    Wh = W5[:, wx:, :]
    Wx = W5[:, :wx, :]
    pad = jnp.zeros((_NMAT, _FP - F, O), W.dtype)
    return jnp.concatenate([Wh, Wx, pad], axis=1).reshape(_NMAT * _FP, O)


def _diffuse(xh, a1, a2):
    def mm(a, v):
        return jax.lax.dot_general(a, v, (((1,), (0,)), ((), ())),
                                   preferred_element_type=jnp.float32)
    k11 = mm(a1, xh)
    k12 = mm(a1, k11)
    k21 = mm(a2, xh)
    k22 = mm(a2, k21)
    return jnp.concatenate([xh, k11, k12, k21, k22], axis=1)


def _mmw(feat, w):
    return jax.lax.dot_general(feat, w, (((1,), (0,)), ((), ())),
                               preferred_element_type=jnp.float32)


def _dcgru_step(xin, wx, h, a1, a2, wg, bg, wc, bc, mask64):
    zpad = jnp.zeros((_N, _FP - _H), jnp.float32)
    if wx == 1:
        base = jnp.concatenate([h, zpad], axis=1)
        xh_g = base + xin * mask64
    else:
        xh_g = jnp.concatenate([h, xin], axis=1)
    g = jax.nn.sigmoid(_mmw(_diffuse(xh_g, a1, a2), wg) + bg)
    r = g[:, :_H]
    u = g[:, _H:]
    rh = r * h
    if wx == 1:
        xh_c = jnp.concatenate([rh, zpad], axis=1) + xin * mask64
    else:
        xh_c = jnp.concatenate([rh, xin], axis=1)
    c = jnp.tanh(_mmw(_diffuse(xh_c, a1, a2), wc) + bc)
    return u * h + (1.0 - u) * c


def _dcrnn_kernel(x_ref, a_ref, m_ref,
                  e0wg_ref, e0bg_ref, e0wc_ref, e0bc_ref,
                  e1wg_ref, e1bg_ref, e1wc_ref, e1bc_ref,
                  d0wg_ref, d0bg_ref, d0wc_ref, d0bc_ref,
                  d1wg_ref, d1bg_ref, d1wc_ref, d1bc_ref,
                  wo_ref, bo_ref, out_ref):
    a1 = a_ref[0]
    a2 = a_ref[1]
    mask64 = m_ref[...]
    e0 = (e0wg_ref[...], e0bg_ref[...], e0wc_ref[...], e0bc_ref[...])
    e1 = (e1wg_ref[...], e1bg_ref[...], e1wc_ref[...], e1bc_ref[...])
    d0 = (d0wg_ref[...], d0bg_ref[...], d0wc_ref[...], d0bc_ref[...])
    d1 = (d1wg_ref[...], d1bg_ref[...], d1wc_ref[...], d1bc_ref[...])

    h0 = jnp.zeros((_N, _H), jnp.float32)
    h1 = jnp.zeros((_N, _H), jnp.float32)
    for t in range(_P):
        x_t = x_ref[0, :, t * 1:t * 1 + 1]
        h0 = _dcgru_step(x_t, 1, h0, a1, a2, *e0, mask64=mask64)
        h1 = _dcgru_step(h0, _H, h1, a1, a2, *e1, mask64=mask64)

    wo = wo_ref[...]
    bo = bo_ref[...]
    xq = jnp.zeros((_N, 1), jnp.float32)
    for q in range(_Q):
        h0 = _dcgru_step(xq, 1, h0, a1, a2, *d0, mask64=mask64)
        h1 = _dcgru_step(h0, _H, h1, a1, a2, *d1, mask64=mask64)
        xq = _mmw(h1, wo) + bo
        out_ref[0, :, q:q + 1] = xq


def kernel(x, As, ycl,
           enc0_Wg, enc0_bg, enc0_Wc, enc0_bc,
           enc1_Wg, enc1_bg, enc1_Wc, enc1_bc,
           dec0_Wg, dec0_bg, dec0_Wc, dec0_bc,
           dec1_Wg, dec1_bg, dec1_Wc, dec1_bc,
           Wo, bo):
    del ycl
    a_hat = _build_adj(As)

    x3 = jnp.transpose(x[..., 0], (0, 2, 1))
    mask64 = jnp.zeros((1, _FP), jnp.float32).at[0, _H].set(1.0)

    ws = [
        _prep_w(enc0_Wg, 1), enc0_bg.reshape(1, -1), _prep_w(enc0_Wc, 1), enc0_bc.reshape(1, -1),
        _prep_w(enc1_Wg, _H), enc1_bg.reshape(1, -1), _prep_w(enc1_Wc, _H), enc1_bc.reshape(1, -1),
        _prep_w(dec0_Wg, 1), dec0_bg.reshape(1, -1), _prep_w(dec0_Wc, 1), dec0_bc.reshape(1, -1),
        _prep_w(dec1_Wg, _H), dec1_bg.reshape(1, -1), _prep_w(dec1_Wc, _H), dec1_bc.reshape(1, -1),
        Wo, bo.reshape(1, 1),
    ]

    full = lambda s: pl.BlockSpec(s, lambda b: tuple(0 for _ in s))
    in_specs = [
        pl.BlockSpec((1, _N, _P), lambda b: (b, 0, 0)),
        full((2, _N, _N)),
        full((1, _FP)),
    ] + [full(w.shape) for w in ws]

    out = pl.pallas_call(
        _dcrnn_kernel,
        grid=(_B,),
        in_specs=in_specs,
        out_specs=pl.BlockSpec((1, _N, _Q), lambda b: (b, 0, 0)),
        out_shape=jax.ShapeDtypeStruct((_B, _N, _Q), jnp.float32),
    )(x3, a_hat, mask64, *ws)

    return jnp.transpose(out, (0, 2, 1))
